# baseline (speedup 1.0000x reference)
import functools
import jax
import jax.numpy as jnp
from jax import lax
from jax.experimental import pallas as pl
from jax.experimental.pallas import tpu as pltpu
from jax.experimental.pallas import tpu_sc as plsc


def _make_gather(S, B, V, D, num_cores, num_subcores):
    NW = num_cores * num_subcores
    N = S * B
    C = 512
    CB = C // 128
    R = D // 8
    n_tasks = N // C
    per_w = n_tasks // NW
    bc_per_s = B // C

    mesh = plsc.VectorSubcoreMesh(core_axis_name="c", subcore_axis_name="s")

    @functools.partial(
        pl.kernel,
        mesh=mesh,
        out_type=jax.ShapeDtypeStruct((S, R, B // 128, 8, 128), jnp.float32),
        scratch_types=[
            [pltpu.VMEM((C,), jnp.int32)] * 2,
            [pltpu.VMEM((C, D), jnp.float32)] * 2,
            [pltpu.VMEM((1, R, CB, 8, 128), jnp.float32)] * 2,
            [pltpu.SemaphoreType.DMA] * 2,
            [pltpu.SemaphoreType.DMA] * 2,
        ],
        compiler_params=pltpu.CompilerParams(
            use_tc_tiling_on_sc=False, needs_layout_passes=False
        ),
    )
    def k(idx_hbm, table_hbm, out_hbm, idx_v, rows_v, tr_v, gsem, wsem):
        wid = lax.axis_index("s") * num_cores + lax.axis_index("c")
        task0 = wid * per_w

        lanes = lax.iota(jnp.int32, 16)

        def load_idx(t, b):
            off = pl.multiple_of(t * C, 8)
            pltpu.sync_copy(idx_hbm.at[pl.ds(off, C)], idx_v[b])

        def gather(t, b):
            return pltpu.async_copy(
                table_hbm.at[idx_v[b]], rows_v[b], gsem[b]
            )

        def wait_gather(t, b):
            pltpu.make_async_copy(
                table_hbm.at[idx_v[b]], rows_v[b], gsem[b]
            ).wait()

        def out_slab(t):
            s = t // bc_per_s
            bc = t % bc_per_s
            return out_hbm.at[
                pl.ds(s, 1), :, pl.ds(pl.multiple_of(bc * CB, CB), CB), :, :
            ]

        def write(t, b):
            return pltpu.async_copy(tr_v[b], out_slab(t), wsem[b])

        def wait_write(t, b):
            pltpu.make_async_copy(tr_v[b], out_slab(t), wsem[b]).wait()

        def transpose(b):
            src = rows_v[b]
            dst = tr_v[b]

            def c_body(c, carry):
                base = c * 128
                for r in range(R):
                    for q in range(8):
                        col = jnp.full((16,), 8 * r + q, jnp.int32)
                        for l0 in range(0, 128, 16):
                            vec = plsc.load_gather(
                                src, (base + l0 + lanes, col)
                            )
                            dst[0, r, c, q, pl.ds(l0, 16)] = vec
                return carry

            lax.fori_loop(0, CB, c_body, 0)

        load_idx(task0, 0)
        gather(task0, 0)
        load_idx(task0 + 1, 1)
        gather(task0 + 1, 1)

        def body(i, carry):
            for b in range(2):
                t = task0 + 2 * i + b
                wait_gather(t, b)

                @pl.when(2 * i + b >= 2)
                def _():
                    wait_write(t - 2, b)

                transpose(b)
                write(t, b)

                @pl.when(2 * i + b + 2 < per_w)
                def _():
                    load_idx(t + 2, b)
                    gather(t + 2, b)

            return carry

        lax.fori_loop(0, per_w // 2, body, 0)

        wait_write(task0 + per_w - 2, 0)
        wait_write(task0 + per_w - 1, 1)

    return k


def kernel(x, table):
    B, S = x.shape
    V, D = table.shape
    info = plsc.get_sparse_core_info()
    k = _make_gather(S, B, V, D, info.num_cores, info.num_subcores)
    xt = x.T.reshape(S * B).astype(jnp.int32)
    out = k(xt, table)
    return out.transpose(2, 4, 0, 1, 3).reshape(B, S, D)


# speedup vs baseline: 1.6656x; 1.6656x over previous
import functools
import jax
import jax.numpy as jnp
from jax import lax
from jax.experimental import pallas as pl
from jax.experimental.pallas import tpu as pltpu
from jax.experimental.pallas import tpu_sc as plsc


def _make_gather(S, B, V, D, num_cores, num_subcores):
    NW = num_cores * num_subcores
    N = S * B
    C = 512
    CB = C // 128
    R = D // 8
    TP = C + 1
    n_tasks = N // C
    per_w = n_tasks // NW
    bc_per_s = B // C

    mesh = plsc.VectorSubcoreMesh(core_axis_name="c", subcore_axis_name="s")

    @functools.partial(
        pl.kernel,
        mesh=mesh,
        out_type=jax.ShapeDtypeStruct((S * R * (B // 128) * 8, 128), jnp.float32),
        scratch_types=[
            [pltpu.VMEM((C,), jnp.int32)] * 2,
            [pltpu.VMEM((C, D), jnp.float32)] * 2,
            [pltpu.VMEM((D, TP), jnp.float32)] * 2,
            [pltpu.SemaphoreType.DMA] * 2,
            [pltpu.SemaphoreType.DMA] * 2,
        ],
        compiler_params=pltpu.CompilerParams(
            use_tc_tiling_on_sc=False, needs_layout_passes=False
        ),
    )
    def k(idx_hbm, table_hbm, out_hbm, idx_v, rows_v, tr_v, gsem, wsem):
        wid = lax.axis_index("s") * num_cores + lax.axis_index("c")
        task0 = wid * per_w

        lanes = lax.iota(jnp.int32, 16)
        lanes_hi = lanes + 16

        def load_idx(t, b):
            off = pl.multiple_of(t * C, 8)
            pltpu.sync_copy(idx_hbm.at[pl.ds(off, C)], idx_v[b])

        def gather(t, b):
            return pltpu.async_copy(
                table_hbm.at[idx_v[b]], rows_v[b], gsem[b]
            )

        def wait_gather(t, b):
            pltpu.make_async_copy(
                table_hbm.at[idx_v[b]], rows_v[b], gsem[b]
            ).wait()

        def write(t, b):
            s = t // bc_per_s
            bc = t % bc_per_s
            for r in range(R):
                for c in range(CB):
                    rc = r * (B // 128) + bc * CB + c
                    dst = out_hbm.at[pl.ds((s * (B // 128) * R + rc) * 8, 8), :]
                    src = tr_v[b].at[pl.ds(8 * r, 8), pl.ds(128 * c, 128)]
                    pltpu.async_copy(src, dst, wsem[b])

        def wait_write(t, b):
            s = t // bc_per_s
            bc = t % bc_per_s
            for r in range(R):
                for c in range(CB):
                    rc = r * (B // 128) + bc * CB + c
                    dst = out_hbm.at[pl.ds((s * (B // 128) * R + rc) * 8, 8), :]
                    src = tr_v[b].at[pl.ds(8 * r, 8), pl.ds(128 * c, 128)]
                    pltpu.make_async_copy(src, dst, wsem[b]).wait()

        def transpose(b):
            src = rows_v[b]
            dst = tr_v[b]

            def j_body(j0, carry):
                for u in range(8):
                    j = j0 * 8 + u
                    col = jnp.full_like(lanes, j)
                    v_lo = plsc.load_gather(src, (col, lanes))
                    v_hi = plsc.load_gather(src, (col, lanes_hi))
                    plsc.store_scatter(dst, (lanes, col), v_lo)
                    plsc.store_scatter(dst, (lanes_hi, col), v_hi)
                return carry

            lax.fori_loop(0, C // 8, j_body, 0)

        load_idx(task0, 0)
        gather(task0, 0)
        load_idx(task0 + 1, 1)
        gather(task0 + 1, 1)

        def body(i, carry):
            for b in range(2):
                t = task0 + 2 * i + b
                wait_gather(t, b)

                @pl.when(2 * i + b >= 2)
                def _():
                    wait_write(t - 2, b)

                transpose(b)
                write(t, b)

                @pl.when(2 * i + b + 2 < per_w)
                def _():
                    load_idx(t + 2, b)
                    gather(t + 2, b)

            return carry

        lax.fori_loop(0, per_w // 2, body, 0)

        wait_write(task0 + per_w - 2, 0)
        wait_write(task0 + per_w - 1, 1)

    return k


def kernel(x, table):
    B, S = x.shape
    V, D = table.shape
    info = plsc.get_sparse_core_info()
    k = _make_gather(S, B, V, D, info.num_cores, info.num_subcores)
    xt = x.T.reshape(S * B).astype(jnp.int32)
    out = k(xt, table)
    out5 = out.reshape(S, D // 8, B // 128, 8, 128)
    return out5.transpose(2, 4, 0, 1, 3).reshape(B, S, D)
